# baseline (speedup 1.0000x reference)
import jax
import jax.numpy as jnp
from jax import lax
from jax.experimental import pallas as pl
from jax.experimental.pallas import tpu as pltpu
from jax.experimental.pallas import tpu_sc as plsc

NUM_USERS = 1000000
NUM_SUBS = 100000
D = 32
B = 16384
K = 20

NC = 2
NS = 16
NW = NC * NS
BPW = B // NW
CHUNK = 64
NCHUNK = BPW // CHUNK
GATHER_N = 128
NEG_PER_CHUNK = CHUNK * K
NEG_GATHERS = NEG_PER_CHUNK // GATHER_N
GROUPS = CHUNK // 16


def _sc_body(uid_hbm, pid_hbm, nid_hbm, user_emb, sub_emb,
             pos_out, neg_out,
             idx_u0, idx_p0, idx_n0, u_v0, sp_v0, sn_v0, pos_v0, neg_v0,
             idx_u1, idx_p1, idx_n1, u_v1, sp_v1, sn_v1, pos_v1, neg_v1,
             sem0, sem1):
    wid = lax.axis_index("s") * NC + lax.axis_index("c")
    bufs = ((idx_u0, idx_p0, idx_n0, u_v0, sp_v0, sn_v0, pos_v0, neg_v0, sem0),
            (idx_u1, idx_p1, idx_n1, u_v1, sp_v1, sn_v1, pos_v1, neg_v1, sem1))

    def stage_and_fire(c):
        idx_u, idx_p, idx_n, u_v, sp_v, sn_v, _, _, sem = bufs[c % 2]
        base = wid * BPW + c * CHUNK
        pltpu.sync_copy(uid_hbm.at[pl.ds(base, CHUNK)], idx_u)
        pltpu.sync_copy(pid_hbm.at[pl.ds(base, CHUNK)], idx_p)
        pltpu.sync_copy(nid_hbm.at[pl.ds(base * K, NEG_PER_CHUNK)], idx_n)
        cps = [pltpu.async_copy(user_emb.at[idx_u], u_v, sem),
               pltpu.async_copy(sub_emb.at[idx_p], sp_v, sem)]
        for j in range(NEG_GATHERS):
            cps.append(pltpu.async_copy(
                sub_emb.at[idx_n.at[pl.ds(j * GATHER_N, GATHER_N)]],
                sn_v.at[pl.ds(j * GATHER_N, GATHER_N), :], sem))
        return cps

    def compute_and_emit(c, cps):
        _, _, _, u_iv, sp_iv, sn_iv, pos_v, neg_v, _ = bufs[c % 2]
        base = wid * BPW + c * CHUNK
        for cp in cps:
            cp.wait()

        WPR = D // 2

        def group_body(g, gcarry):
            rows = g * 16 + lax.iota(jnp.int32, 16)
            rows_k = rows * K
            zero = jnp.zeros((16,), jnp.float32)

            lanes = lax.iota(jnp.int32, 16)

            def unpack2(w):
                lo = plsc.bitcast(w << 16, jnp.float32)
                hi = plsc.bitcast(w & jnp.int32(-65536), jnp.float32)
                return lo, hi

            def t_body(t, accs):
                wdiag = (lanes + t) & (WPR - 1)
                u_lo, u_hi = unpack2(plsc.load_gather(u_iv, [rows, wdiag]))
                p_lo, p_hi = unpack2(plsc.load_gather(sp_iv, [rows, wdiag]))
                out = [accs[0] + u_lo * p_lo + u_hi * p_hi]
                for k in range(K):
                    n_lo, n_hi = unpack2(plsc.load_gather(
                        sn_iv, [rows_k + k, wdiag]))
                    out.append(accs[k + 1] + u_lo * n_lo + u_hi * n_hi)
                return tuple(out)

            accs = lax.fori_loop(0, WPR, t_body, (zero,) * (K + 1))
            pos_v[pl.ds(g * 16, 16)] = accs[0]
            for k in range(K):
                plsc.store_scatter(neg_v, [rows_k + k], accs[k + 1])
            return gcarry

        lax.fori_loop(0, GROUPS, group_body, 0)

        pltpu.sync_copy(pos_v, pos_out.at[pl.ds(base, CHUNK)])
        pltpu.sync_copy(neg_v, neg_out.at[pl.ds(base * K, NEG_PER_CHUNK)])

    inflight = stage_and_fire(0)
    for c in range(NCHUNK):
        nxt = stage_and_fire(c + 1) if c + 1 < NCHUNK else None
        compute_and_emit(c, inflight)
        inflight = nxt


@jax.jit
def _sc_forward(user_ids, pos_sub_ids, neg_ids_2d, user_emb, sub_emb):
    mesh = plsc.VectorSubcoreMesh(core_axis_name="c", subcore_axis_name="s")
    return pl.kernel(
        _sc_body,
        out_type=(jax.ShapeDtypeStruct((B,), jnp.float32),
                  jax.ShapeDtypeStruct((B * K,), jnp.float32)),
        mesh=mesh,
        scratch_types=[
            pltpu.VMEM((CHUNK,), jnp.int32),
            pltpu.VMEM((CHUNK,), jnp.int32),
            pltpu.VMEM((NEG_PER_CHUNK,), jnp.int32),
            pltpu.VMEM((CHUNK, D // 2), jnp.int32),
            pltpu.VMEM((CHUNK, D // 2), jnp.int32),
            pltpu.VMEM((NEG_PER_CHUNK, D // 2), jnp.int32),
            pltpu.VMEM((CHUNK,), jnp.float32),
            pltpu.VMEM((NEG_PER_CHUNK,), jnp.float32),
            pltpu.VMEM((CHUNK,), jnp.int32),
            pltpu.VMEM((CHUNK,), jnp.int32),
            pltpu.VMEM((NEG_PER_CHUNK,), jnp.int32),
            pltpu.VMEM((CHUNK, D // 2), jnp.int32),
            pltpu.VMEM((CHUNK, D // 2), jnp.int32),
            pltpu.VMEM((NEG_PER_CHUNK, D // 2), jnp.int32),
            pltpu.VMEM((CHUNK,), jnp.float32),
            pltpu.VMEM((NEG_PER_CHUNK,), jnp.float32),
            pltpu.SemaphoreType.DMA,
            pltpu.SemaphoreType.DMA,
        ],
        compiler_params=pltpu.CompilerParams(use_tc_tiling_on_sc=False, needs_layout_passes=False),
    )(user_ids, pos_sub_ids, neg_ids_2d, user_emb, sub_emb)


def kernel(user_ids, pos_sub_ids, neg_sub_ids, user_emb, sub_emb):
    uid = user_ids.astype(jnp.int32)
    pid = pos_sub_ids.astype(jnp.int32)
    nid = neg_sub_ids.astype(jnp.int32).reshape(B * K)
    ue = jax.lax.bitcast_convert_type(
        user_emb.astype(jnp.bfloat16).reshape(NUM_USERS * D // 2, 2),
        jnp.int32).reshape(NUM_USERS, D // 2)
    se = jax.lax.bitcast_convert_type(
        sub_emb.astype(jnp.bfloat16).reshape(NUM_SUBS * D // 2, 2),
        jnp.int32).reshape(NUM_SUBS, D // 2)
    pos_flat, neg_flat = _sc_forward(uid, pid, nid, ue, se)
    return (pos_flat, neg_flat.reshape(B, K))


# speedup vs baseline: 14.4816x; 14.4816x over previous
import jax
import jax.numpy as jnp
from jax import lax
from jax.experimental import pallas as pl
from jax.experimental.pallas import tpu as pltpu
from jax.experimental.pallas import tpu_sc as plsc

NUM_USERS = 1000000
NUM_SUBS = 100000
D = 32
B = 16384
K = 20

NC = 2
NS = 16
NW = NC * NS
BPW = B // NW
CHUNK = 64
NCHUNK = BPW // CHUNK
GATHER_N = 128
NEG_PER_CHUNK = CHUNK * K
NEG_GATHERS = NEG_PER_CHUNK // GATHER_N
GROUPS = CHUNK // 16


def _sc_body(uid_hbm, pid_hbm, nid_hbm, user_emb, sub_emb,
             pos_out, neg_out,
             idx_u0, idx_p0, idx_n0, u_v0, sp_v0, sn_v0, pos_v0, neg_v0,
             idx_u1, idx_p1, idx_n1, u_v1, sp_v1, sn_v1, pos_v1, neg_v1,
             sem0, sem1, out_sem0, out_sem1):
    out_sems = (out_sem0, out_sem1)
    wid = lax.axis_index("s") * NC + lax.axis_index("c")
    bufs = ((idx_u0, idx_p0, idx_n0, u_v0, sp_v0, sn_v0, pos_v0, neg_v0, sem0),
            (idx_u1, idx_p1, idx_n1, u_v1, sp_v1, sn_v1, pos_v1, neg_v1, sem1))

    def stage_ids(c, sem):
        idx_u, idx_p, idx_n = bufs[c % 2][:3]
        base = wid * BPW + c * CHUNK
        return [pltpu.async_copy(uid_hbm.at[pl.ds(base, CHUNK)], idx_u, sem),
                pltpu.async_copy(pid_hbm.at[pl.ds(base, CHUNK)], idx_p, sem),
                pltpu.async_copy(nid_hbm.at[pl.ds(base * K, NEG_PER_CHUNK)],
                                 idx_n, sem)]

    def fire(c):
        idx_u, idx_p, idx_n, u_v, sp_v, sn_v, _, _, sem = bufs[c % 2]
        cps = [pltpu.async_copy(user_emb.at[idx_u], u_v, sem),
               pltpu.async_copy(sub_emb.at[idx_p], sp_v, sem)]
        for j in range(NEG_GATHERS):
            cps.append(pltpu.async_copy(
                sub_emb.at[idx_n.at[pl.ds(j * GATHER_N, GATHER_N)]],
                sn_v.at[pl.ds(j * GATHER_N, GATHER_N), :], sem))
        return cps

    def compute_and_emit(c, out_sem):
        _, _, _, u_v, sp_v, sn_v, pos_v, neg_v, _ = bufs[c % 2]
        base = wid * BPW + c * CHUNK

        def group_body(g, gcarry):
            rows = g * 16 + lax.iota(jnp.int32, 16)
            rows_k = rows * K
            zero = jnp.zeros((16,), jnp.float32)

            lanes = lax.iota(jnp.int32, 16)

            def d_body(d, accs):
                cold = (lanes + d) & (D - 1)
                u_d = plsc.load_gather(u_v, [rows, cold])
                p_d = plsc.load_gather(sp_v, [rows, cold])
                out = [accs[0] + u_d * p_d]
                for k in range(K):
                    n_d = plsc.load_gather(sn_v, [rows_k + k, cold])
                    out.append(accs[k + 1] + u_d * n_d)
                return tuple(out)

            accs = lax.fori_loop(0, D, d_body, (zero,) * (K + 1))
            pos_v[pl.ds(g * 16, 16)] = accs[0]
            for k in range(K):
                plsc.store_scatter(neg_v, [rows_k + k], accs[k + 1])
            return gcarry

        lax.fori_loop(0, GROUPS, group_body, 0)

        return [pltpu.async_copy(pos_v, pos_out.at[pl.ds(base, CHUNK)],
                                 out_sem),
                pltpu.async_copy(neg_v, neg_out.at[pl.ds(base * K,
                                                         NEG_PER_CHUNK)],
                                 out_sem)]

    ids_cps = {0: stage_ids(0, sem0), 1: stage_ids(1, sem1)}
    for cp in ids_cps[0]:
        cp.wait()
    gat_cps = {0: fire(0)}
    out_cps = {}
    for c in range(NCHUNK):
        if c + 1 < NCHUNK:
            for cp in ids_cps[c + 1]:
                cp.wait()
            gat_cps[c + 1] = fire(c + 1)
        for cp in gat_cps[c]:
            cp.wait()
        if c + 2 < NCHUNK:
            ids_cps[c + 2] = stage_ids(c + 2, bufs[c % 2][8])
        if c - 2 >= 0:
            for cp in out_cps[c - 2]:
                cp.wait()
        out_cps[c] = compute_and_emit(c, out_sems[c % 2])
    for c in (NCHUNK - 2, NCHUNK - 1):
        for cp in out_cps[c]:
            cp.wait()


@jax.jit
def _sc_forward(user_ids, pos_sub_ids, neg_ids_2d, user_emb, sub_emb):
    mesh = plsc.VectorSubcoreMesh(core_axis_name="c", subcore_axis_name="s")
    return pl.kernel(
        _sc_body,
        out_type=(jax.ShapeDtypeStruct((B,), jnp.float32),
                  jax.ShapeDtypeStruct((B * K,), jnp.float32)),
        mesh=mesh,
        scratch_types=[
            pltpu.VMEM((CHUNK,), jnp.int32),
            pltpu.VMEM((CHUNK,), jnp.int32),
            pltpu.VMEM((NEG_PER_CHUNK,), jnp.int32),
            pltpu.VMEM((CHUNK, D), jnp.float32),
            pltpu.VMEM((CHUNK, D), jnp.float32),
            pltpu.VMEM((NEG_PER_CHUNK, D), jnp.float32),
            pltpu.VMEM((CHUNK,), jnp.float32),
            pltpu.VMEM((NEG_PER_CHUNK,), jnp.float32),
            pltpu.VMEM((CHUNK,), jnp.int32),
            pltpu.VMEM((CHUNK,), jnp.int32),
            pltpu.VMEM((NEG_PER_CHUNK,), jnp.int32),
            pltpu.VMEM((CHUNK, D), jnp.float32),
            pltpu.VMEM((CHUNK, D), jnp.float32),
            pltpu.VMEM((NEG_PER_CHUNK, D), jnp.float32),
            pltpu.VMEM((CHUNK,), jnp.float32),
            pltpu.VMEM((NEG_PER_CHUNK,), jnp.float32),
            pltpu.SemaphoreType.DMA,
            pltpu.SemaphoreType.DMA,
            pltpu.SemaphoreType.DMA,
            pltpu.SemaphoreType.DMA,
        ],
        compiler_params=pltpu.CompilerParams(use_tc_tiling_on_sc=False, needs_layout_passes=False),
    )(user_ids, pos_sub_ids, neg_ids_2d, user_emb, sub_emb)


def kernel(user_ids, pos_sub_ids, neg_sub_ids, user_emb, sub_emb):
    uid = user_ids.astype(jnp.int32)
    pid = pos_sub_ids.astype(jnp.int32)
    nid = neg_sub_ids.astype(jnp.int32).reshape(B * K)
    pos_flat, neg_flat = _sc_forward(uid, pid, nid, user_emb, sub_emb)
    return (pos_flat, neg_flat.reshape(B, K))
